# Initial kernel scaffold; baseline (speedup 1.0000x reference)
import jax
import jax.numpy as jnp
from jax import lax
from jax.experimental import pallas as pl
from jax.experimental.pallas import tpu as pltpu
from jax.experimental.pallas import tpu_sc as plsc

N_NODES = 10000
N_EDGES = 320000
D_FEAT = 128
D_AUG = 144
D_HID = 256
NC = 2
NS = 16
CH = 80
RPT = N_NODES // NS
RB = 2000

_f32 = jnp.float32


def _edge_pass(tbl_hbm, src_hbm, dst_hbm, acc_sh, srcv, dstv, rows, sem,
               base, n_chunks):
    @pl.loop(0, n_chunks)
    def _(i):
        off = base + i * CH
        pltpu.sync_copy(src_hbm.at[pl.ds(off, CH)], srcv)
        pltpu.sync_copy(dst_hbm.at[pl.ds(off, CH)], dstv)
        pltpu.async_copy(tbl_hbm.at[srcv], rows, sem).wait()
        pltpu.sync_copy(rows, acc_sh.at[dstv], add=True)


def _agg1_body(xa_hbm, src_hbm, dst_hbm, zer_hbm, outa, outb,
               acc, srcv, dstv, rows, sem):
    c = lax.axis_index("c")
    s = lax.axis_index("s")
    r0 = s * RPT
    pltpu.sync_copy(zer_hbm.at[pl.ds(r0, RPT)], acc.at[pl.ds(r0, RPT)])
    plsc.subcore_barrier()

    epw = N_EDGES // (NC * NS)
    base = (c * NS + s) * epw
    _edge_pass(xa_hbm, src_hbm, dst_hbm, acc, srcv, dstv, rows, sem,
               base, epw // CH)
    plsc.subcore_barrier()

    @pl.when(c == 0)
    def _():
        pltpu.sync_copy(acc.at[pl.ds(r0, RPT)], outa.at[pl.ds(r0, RPT)])

    @pl.when(c == 1)
    def _():
        pltpu.sync_copy(acc.at[pl.ds(r0, RPT)], outb.at[pl.ds(r0, RPT)])


def _agg2_body(h1a_hbm, h1b_hbm, src_hbm, dst_hbm, zer_hbm, outa, outb,
               acc, srcv, dstv, rows, sem):
    c = lax.axis_index("c")
    s = lax.axis_index("s")
    r0 = s * RPT
    pltpu.sync_copy(zer_hbm.at[pl.ds(r0, RPT)], acc.at[pl.ds(r0, RPT)])
    plsc.subcore_barrier()

    epw = N_EDGES // NS
    base = s * epw

    @pl.when(c == 0)
    def _():
        _edge_pass(h1a_hbm, src_hbm, dst_hbm, acc, srcv, dstv, rows, sem,
                   base, epw // CH)

    @pl.when(c == 1)
    def _():
        _edge_pass(h1b_hbm, src_hbm, dst_hbm, acc, srcv, dstv, rows, sem,
                   base, epw // CH)

    plsc.subcore_barrier()

    @pl.when(c == 0)
    def _():
        pltpu.sync_copy(acc.at[pl.ds(r0, RPT)], outa.at[pl.ds(r0, RPT)])

    @pl.when(c == 1)
    def _():
        pltpu.sync_copy(acc.at[pl.ds(r0, RPT)], outb.at[pl.ds(r0, RPT)])


def _dot(a, b):
    return jnp.dot(a, b, preferred_element_type=_f32,
                   precision=lax.Precision.HIGHEST)


def _l1_body(pa, pb, x, wl, wr, b, ha, hb, rinv):
    t = pa[...] + pb[...]
    cnt = t[:, D_FEAT:D_FEAT + 1]
    r = 1.0 / jnp.maximum(cnt, 1.0)
    agg = t[:, :D_FEAT] * r
    z = _dot(agg, wl[...]) + _dot(x[...], wr[...]) + b[...]
    h = jnp.maximum(z, 0.0)
    ha[...] = h[:, :D_FEAT]
    hb[...] = h[:, D_FEAT:]
    rinv[...] = r


def _l2_body(s2a, s2b, ha, hb, rinv, wl0, wl1, wr0, wr1, b2, wfc, bfc, out):
    r = rinv[...]
    z = (_dot(s2a[...] * r, wl0[...]) + _dot(s2b[...] * r, wl1[...])
         + _dot(ha[...], wr0[...]) + _dot(hb[...], wr1[...]) + b2[...])
    h2 = jnp.maximum(z, 0.0)
    out[...] = _dot(h2, wfc[...]) + bfc[0, 0]


def kernel(x, edge_index, W1l, b1l, W1r, W2l, b2l, W2r, Wfc, bfc):
    src = edge_index[0].astype(jnp.int32)
    dst = edge_index[1].astype(jnp.int32)
    xa = jnp.concatenate(
        [x, jnp.ones((N_NODES, D_AUG - D_FEAT), _f32)], axis=1)
    z144 = jnp.zeros((N_NODES, D_AUG), _f32)
    z128 = jnp.zeros((N_NODES, D_FEAT), _f32)

    mesh = plsc.VectorSubcoreMesh(core_axis_name="c", subcore_axis_name="s")

    agg1 = pl.kernel(
        _agg1_body,
        out_type=[jax.ShapeDtypeStruct((N_NODES, D_AUG), _f32),
                  jax.ShapeDtypeStruct((N_NODES, D_AUG), _f32)],
        mesh=mesh,
        scratch_types=[pltpu.VMEM_SHARED((N_NODES, D_AUG), _f32),
                       pltpu.VMEM((CH,), jnp.int32),
                       pltpu.VMEM((CH,), jnp.int32),
                       pltpu.VMEM((CH, D_AUG), _f32),
                       pltpu.SemaphoreType.DMA],
    )
    pa, pb = agg1(xa, src, dst, z144)

    grid1 = (N_NODES // RB,)
    ha, hb, rinv = pl.pallas_call(
        _l1_body,
        grid=grid1,
        in_specs=[
            pl.BlockSpec((RB, D_AUG), lambda i: (i, 0)),
            pl.BlockSpec((RB, D_AUG), lambda i: (i, 0)),
            pl.BlockSpec((RB, D_FEAT), lambda i: (i, 0)),
            pl.BlockSpec((D_FEAT, D_HID), lambda i: (0, 0)),
            pl.BlockSpec((D_FEAT, D_HID), lambda i: (0, 0)),
            pl.BlockSpec((1, D_HID), lambda i: (0, 0)),
        ],
        out_specs=[
            pl.BlockSpec((RB, D_FEAT), lambda i: (i, 0)),
            pl.BlockSpec((RB, D_FEAT), lambda i: (i, 0)),
            pl.BlockSpec((RB, 1), lambda i: (i, 0)),
        ],
        out_shape=[jax.ShapeDtypeStruct((N_NODES, D_FEAT), _f32),
                   jax.ShapeDtypeStruct((N_NODES, D_FEAT), _f32),
                   jax.ShapeDtypeStruct((N_NODES, 1), _f32)],
    )(pa, pb, x, W1l.T, W1r.T, b1l[None, :])

    agg2 = pl.kernel(
        _agg2_body,
        out_type=[jax.ShapeDtypeStruct((N_NODES, D_FEAT), _f32),
                  jax.ShapeDtypeStruct((N_NODES, D_FEAT), _f32)],
        mesh=mesh,
        scratch_types=[pltpu.VMEM_SHARED((N_NODES, D_FEAT), _f32),
                       pltpu.VMEM((CH,), jnp.int32),
                       pltpu.VMEM((CH,), jnp.int32),
                       pltpu.VMEM((CH, D_FEAT), _f32),
                       pltpu.SemaphoreType.DMA],
    )
    s2a, s2b = agg2(ha, hb, src, dst, z128)

    w2lT = W2l.T
    w2rT = W2r.T
    out = pl.pallas_call(
        _l2_body,
        grid=grid1,
        in_specs=[
            pl.BlockSpec((RB, D_FEAT), lambda i: (i, 0)),
            pl.BlockSpec((RB, D_FEAT), lambda i: (i, 0)),
            pl.BlockSpec((RB, D_FEAT), lambda i: (i, 0)),
            pl.BlockSpec((RB, D_FEAT), lambda i: (i, 0)),
            pl.BlockSpec((RB, 1), lambda i: (i, 0)),
            pl.BlockSpec((D_FEAT, D_HID), lambda i: (0, 0)),
            pl.BlockSpec((D_FEAT, D_HID), lambda i: (0, 0)),
            pl.BlockSpec((D_FEAT, D_HID), lambda i: (0, 0)),
            pl.BlockSpec((D_FEAT, D_HID), lambda i: (0, 0)),
            pl.BlockSpec((1, D_HID), lambda i: (0, 0)),
            pl.BlockSpec((D_HID, 1), lambda i: (0, 0)),
            pl.BlockSpec((1, 1), lambda i: (0, 0)),
        ],
        out_specs=[pl.BlockSpec((RB, 1), lambda i: (i, 0))],
        out_shape=[jax.ShapeDtypeStruct((N_NODES, 1), _f32)],
    )(s2a, s2b, ha, hb, rinv,
      w2lT[:D_FEAT], w2lT[D_FEAT:], w2rT[:D_FEAT], w2rT[D_FEAT:],
      b2l[None, :], Wfc.T, bfc[None, :])[0]

    return out


# speedup vs baseline: 4.1143x; 4.1143x over previous
import jax
import jax.numpy as jnp
from jax import lax
from jax.experimental import pallas as pl
from jax.experimental.pallas import tpu as pltpu
from jax.experimental.pallas import tpu_sc as plsc

N_NODES = 10000
N_PAD = 10240
N_EDGES = 320000
D_FEAT = 128
D_AUG = 144
D_HID = 256
NC = 2
NS = 16
CH = 80
RPT = N_PAD // NS
RB = 2048

_f32 = jnp.float32


def _edge_pass(tbl_hbm, src_hbm, dst_hbm, acc_sh, srcv, dstv, rows, sem,
               base, n_chunks):
    @pl.loop(0, n_chunks)
    def _(i):
        off = base + i * CH
        pltpu.sync_copy(src_hbm.at[pl.ds(off, CH)], srcv)
        pltpu.sync_copy(dst_hbm.at[pl.ds(off, CH)], dstv)
        pltpu.async_copy(tbl_hbm.at[srcv], rows, sem).wait()
        pltpu.sync_copy(rows, acc_sh.at[dstv], add=True)


def _agg1_body(xa_hbm, src_hbm, dst_hbm, zer_hbm, outa, outb,
               acc, srcv, dstv, rows, sem):
    c = lax.axis_index("c")
    s = lax.axis_index("s")
    r0 = s * RPT
    pltpu.sync_copy(zer_hbm.at[pl.ds(r0, RPT)], acc.at[pl.ds(r0, RPT)])
    plsc.subcore_barrier()

    epw = N_EDGES // (NC * NS)
    base = (c * NS + s) * epw
    _edge_pass(xa_hbm, src_hbm, dst_hbm, acc, srcv, dstv, rows, sem,
               base, epw // CH)
    plsc.subcore_barrier()

    @pl.when(c == 0)
    def _():
        pltpu.sync_copy(acc.at[pl.ds(r0, RPT)], outa.at[pl.ds(r0, RPT)])

    @pl.when(c == 1)
    def _():
        pltpu.sync_copy(acc.at[pl.ds(r0, RPT)], outb.at[pl.ds(r0, RPT)])


def _agg2_body(h1a_hbm, h1b_hbm, src_hbm, dst_hbm, zer_hbm, outa, outb,
               acc, srcv, dstv, rows, sem):
    c = lax.axis_index("c")
    s = lax.axis_index("s")
    r0 = s * RPT
    pltpu.sync_copy(zer_hbm.at[pl.ds(r0, RPT)], acc.at[pl.ds(r0, RPT)])
    plsc.subcore_barrier()

    epw = N_EDGES // NS
    base = s * epw

    @pl.when(c == 0)
    def _():
        _edge_pass(h1a_hbm, src_hbm, dst_hbm, acc, srcv, dstv, rows, sem,
                   base, epw // CH)

    @pl.when(c == 1)
    def _():
        _edge_pass(h1b_hbm, src_hbm, dst_hbm, acc, srcv, dstv, rows, sem,
                   base, epw // CH)

    plsc.subcore_barrier()

    @pl.when(c == 0)
    def _():
        pltpu.sync_copy(acc.at[pl.ds(r0, RPT)], outa.at[pl.ds(r0, RPT)])

    @pl.when(c == 1)
    def _():
        pltpu.sync_copy(acc.at[pl.ds(r0, RPT)], outb.at[pl.ds(r0, RPT)])


def _dot(a, b):
    return jnp.dot(a, b, preferred_element_type=_f32,
                   precision=lax.Precision.HIGHEST)


def _l1_body(pa, pb, x, wl, wr, b, ha, hb, rinv):
    t = pa[...] + pb[...]
    cnt = t[:, D_FEAT:D_FEAT + 1]
    r = 1.0 / jnp.maximum(cnt, 1.0)
    agg = t[:, :D_FEAT] * r
    z = _dot(agg, wl[...]) + _dot(x[...], wr[...]) + b[...]
    h = jnp.maximum(z, 0.0)
    ha[...] = h[:, :D_FEAT]
    hb[...] = h[:, D_FEAT:]
    rinv[...] = r


def _l2_body(s2a, s2b, ha, hb, rinv, wl0, wl1, wr0, wr1, b2, wfc, bfc, out):
    r = rinv[...]
    z = (_dot(s2a[...] * r, wl0[...]) + _dot(s2b[...] * r, wl1[...])
         + _dot(ha[...], wr0[...]) + _dot(hb[...], wr1[...]) + b2[...])
    h2 = jnp.maximum(z, 0.0)
    out[...] = _dot(h2, wfc[...]) + bfc[...]


def kernel(x, edge_index, W1l, b1l, W1r, W2l, b2l, W2r, Wfc, bfc):
    src = edge_index[0].astype(jnp.int32)
    dst = edge_index[1].astype(jnp.int32)
    xp = jnp.pad(x, ((0, N_PAD - N_NODES), (0, 0)))
    xa = jnp.concatenate(
        [xp, jnp.ones((N_PAD, D_AUG - D_FEAT), _f32)], axis=1)
    z144 = jnp.zeros((N_PAD, D_AUG), _f32)
    z128 = jnp.zeros((N_PAD, D_FEAT), _f32)

    mesh = plsc.VectorSubcoreMesh(core_axis_name="c", subcore_axis_name="s")
    sc_params = pltpu.CompilerParams(use_tc_tiling_on_sc=False)

    agg1 = pl.kernel(
        _agg1_body,
        out_type=[jax.ShapeDtypeStruct((N_PAD, D_AUG), _f32),
                  jax.ShapeDtypeStruct((N_PAD, D_AUG), _f32)],
        mesh=mesh,
        scratch_types=[pltpu.VMEM_SHARED((N_PAD, D_AUG), _f32),
                       pltpu.VMEM((CH,), jnp.int32),
                       pltpu.VMEM((CH,), jnp.int32),
                       pltpu.VMEM((CH, D_AUG), _f32),
                       pltpu.SemaphoreType.DMA],
        compiler_params=sc_params,
    )
    pa, pb = agg1(xa, src, dst, z144)

    grid1 = (N_PAD // RB,)
    ha, hb, rinv = pl.pallas_call(
        _l1_body,
        grid=grid1,
        in_specs=[
            pl.BlockSpec((RB, D_AUG), lambda i: (i, 0)),
            pl.BlockSpec((RB, D_AUG), lambda i: (i, 0)),
            pl.BlockSpec((RB, D_FEAT), lambda i: (i, 0)),
            pl.BlockSpec((D_FEAT, D_HID), lambda i: (0, 0)),
            pl.BlockSpec((D_FEAT, D_HID), lambda i: (0, 0)),
            pl.BlockSpec((1, D_HID), lambda i: (0, 0)),
        ],
        out_specs=[
            pl.BlockSpec((RB, D_FEAT), lambda i: (i, 0)),
            pl.BlockSpec((RB, D_FEAT), lambda i: (i, 0)),
            pl.BlockSpec((RB, 1), lambda i: (i, 0)),
        ],
        out_shape=[jax.ShapeDtypeStruct((N_PAD, D_FEAT), _f32),
                   jax.ShapeDtypeStruct((N_PAD, D_FEAT), _f32),
                   jax.ShapeDtypeStruct((N_PAD, 1), _f32)],
    )(pa, pb, xp, W1l.T, W1r.T, b1l[None, :])

    agg2 = pl.kernel(
        _agg2_body,
        out_type=[jax.ShapeDtypeStruct((N_PAD, D_FEAT), _f32),
                  jax.ShapeDtypeStruct((N_PAD, D_FEAT), _f32)],
        mesh=mesh,
        scratch_types=[pltpu.VMEM_SHARED((N_PAD, D_FEAT), _f32),
                       pltpu.VMEM((CH,), jnp.int32),
                       pltpu.VMEM((CH,), jnp.int32),
                       pltpu.VMEM((CH, D_FEAT), _f32),
                       pltpu.SemaphoreType.DMA],
        compiler_params=sc_params,
    )
    s2a, s2b = agg2(ha, hb, src, dst, z128)

    w2lT = W2l.T
    w2rT = W2r.T
    out = pl.pallas_call(
        _l2_body,
        grid=grid1,
        in_specs=[
            pl.BlockSpec((RB, D_FEAT), lambda i: (i, 0)),
            pl.BlockSpec((RB, D_FEAT), lambda i: (i, 0)),
            pl.BlockSpec((RB, D_FEAT), lambda i: (i, 0)),
            pl.BlockSpec((RB, D_FEAT), lambda i: (i, 0)),
            pl.BlockSpec((RB, 1), lambda i: (i, 0)),
            pl.BlockSpec((D_FEAT, D_HID), lambda i: (0, 0)),
            pl.BlockSpec((D_FEAT, D_HID), lambda i: (0, 0)),
            pl.BlockSpec((D_FEAT, D_HID), lambda i: (0, 0)),
            pl.BlockSpec((D_FEAT, D_HID), lambda i: (0, 0)),
            pl.BlockSpec((1, D_HID), lambda i: (0, 0)),
            pl.BlockSpec((D_HID, 1), lambda i: (0, 0)),
            pl.BlockSpec((1, 1), lambda i: (0, 0)),
        ],
        out_specs=[pl.BlockSpec((RB, 1), lambda i: (i, 0))],
        out_shape=[jax.ShapeDtypeStruct((N_PAD, 1), _f32)],
    )(s2a, s2b, ha, hb, rinv,
      w2lT[:D_FEAT], w2lT[D_FEAT:], w2rT[:D_FEAT], w2rT[D_FEAT:],
      b2l[None, :], Wfc.T, bfc[None, :])[0]

    return out[:N_NODES]


# speedup vs baseline: 7.6694x; 1.8641x over previous
import jax
import jax.numpy as jnp
from jax import lax
from jax.experimental import pallas as pl
from jax.experimental.pallas import tpu as pltpu
from jax.experimental.pallas import tpu_sc as plsc

N_NODES = 10000
N_PAD = 10240
N_EDGES = 320000
D_FEAT = 128
D_AUG = 144
D_HID = 256
NC = 2
NS = 16
CH = 80
RPT = N_PAD // NS
RB = 2048

_f32 = jnp.float32


def _edge_pass(tbl_hbm, eidx_hbm, acc_sh, idxs, rowss, sems, base, n_chunks):
    def issue(i, b):
        off = base + i * CH
        pltpu.sync_copy(eidx_hbm.at[:, pl.ds(off, CH)], idxs[b])
        pltpu.async_copy(tbl_hbm.at[idxs[b].at[0]], rowss[b], sems[b])

    def finish(b):
        pltpu.make_async_copy(tbl_hbm.at[idxs[b].at[0]], rowss[b],
                              sems[b]).wait()
        pltpu.sync_copy(rowss[b], acc_sh.at[idxs[b].at[1]], add=True)

    issue(0, 0)

    @pl.loop(0, (n_chunks - 1) // 2)
    def _(g):
        i = 2 * g
        issue(i + 1, 1)
        finish(0)
        issue(i + 2, 0)
        finish(1)

    if n_chunks % 2 == 1:
        finish(0)
    else:
        issue(n_chunks - 1, 1)
        finish(0)
        finish(1)


def _agg1_body(xa_hbm, eidx_hbm, zer_hbm, outa, outb,
               acc, idx0, idx1, rows0, rows1, sem0, sem1):
    c = lax.axis_index("c")
    s = lax.axis_index("s")
    r0 = s * RPT
    pltpu.sync_copy(zer_hbm.at[pl.ds(r0, RPT)], acc.at[pl.ds(r0, RPT)])
    plsc.subcore_barrier()

    epw = N_EDGES // (NC * NS)
    base = (c * NS + s) * epw
    _edge_pass(xa_hbm, eidx_hbm, acc, (idx0, idx1), (rows0, rows1),
               (sem0, sem1), base, epw // CH)
    plsc.subcore_barrier()

    @pl.when(c == 0)
    def _():
        pltpu.sync_copy(acc.at[pl.ds(r0, RPT)], outa.at[pl.ds(r0, RPT)])

    @pl.when(c == 1)
    def _():
        pltpu.sync_copy(acc.at[pl.ds(r0, RPT)], outb.at[pl.ds(r0, RPT)])


def _agg2_body(h1a_hbm, h1b_hbm, eidx_hbm, zer_hbm, outa, outb,
               acc, idx0, idx1, rows0, rows1, sem0, sem1):
    c = lax.axis_index("c")
    s = lax.axis_index("s")
    r0 = s * RPT
    pltpu.sync_copy(zer_hbm.at[pl.ds(r0, RPT)], acc.at[pl.ds(r0, RPT)])
    plsc.subcore_barrier()

    epw = N_EDGES // NS
    base = s * epw

    @pl.when(c == 0)
    def _():
        _edge_pass(h1a_hbm, eidx_hbm, acc, (idx0, idx1), (rows0, rows1),
                   (sem0, sem1), base, epw // CH)

    @pl.when(c == 1)
    def _():
        _edge_pass(h1b_hbm, eidx_hbm, acc, (idx0, idx1), (rows0, rows1),
                   (sem0, sem1), base, epw // CH)

    plsc.subcore_barrier()

    @pl.when(c == 0)
    def _():
        pltpu.sync_copy(acc.at[pl.ds(r0, RPT)], outa.at[pl.ds(r0, RPT)])

    @pl.when(c == 1)
    def _():
        pltpu.sync_copy(acc.at[pl.ds(r0, RPT)], outb.at[pl.ds(r0, RPT)])


def _dot(a, b):
    return jnp.dot(a, b, preferred_element_type=_f32,
                   precision=lax.Precision.HIGHEST)


def _l1_body(pa, pb, x, wl, wr, b, ha, hb, rinv):
    t = pa[...] + pb[...]
    cnt = t[:, D_FEAT:D_FEAT + 1]
    r = 1.0 / jnp.maximum(cnt, 1.0)
    agg = t[:, :D_FEAT] * r
    z = _dot(agg, wl[...]) + _dot(x[...], wr[...]) + b[...]
    h = jnp.maximum(z, 0.0)
    ha[...] = h[:, :D_FEAT]
    hb[...] = h[:, D_FEAT:]
    rinv[...] = r


def _l2_body(s2a, s2b, ha, hb, rinv, wl0, wl1, wr0, wr1, b2, wfc, bfc, out):
    r = rinv[...]
    z = (_dot(s2a[...] * r, wl0[...]) + _dot(s2b[...] * r, wl1[...])
         + _dot(ha[...], wr0[...]) + _dot(hb[...], wr1[...]) + b2[...])
    h2 = jnp.maximum(z, 0.0)
    out[...] = _dot(h2, wfc[...]) + bfc[...]


def kernel(x, edge_index, W1l, b1l, W1r, W2l, b2l, W2r, Wfc, bfc):
    eidx = edge_index.astype(jnp.int32)
    xp = jnp.pad(x, ((0, N_PAD - N_NODES), (0, 0)))
    xa = jnp.concatenate(
        [xp, jnp.ones((N_PAD, D_AUG - D_FEAT), _f32)], axis=1)
    z144 = jnp.zeros((N_PAD, D_AUG), _f32)
    z128 = jnp.zeros((N_PAD, D_FEAT), _f32)

    mesh = plsc.VectorSubcoreMesh(core_axis_name="c", subcore_axis_name="s")
    sc_params = pltpu.CompilerParams(use_tc_tiling_on_sc=False)

    agg1 = pl.kernel(
        _agg1_body,
        out_type=[jax.ShapeDtypeStruct((N_PAD, D_AUG), _f32),
                  jax.ShapeDtypeStruct((N_PAD, D_AUG), _f32)],
        mesh=mesh,
        scratch_types=[pltpu.VMEM_SHARED((N_PAD, D_AUG), _f32),
                       pltpu.VMEM((2, CH), jnp.int32),
                       pltpu.VMEM((2, CH), jnp.int32),
                       pltpu.VMEM((CH, D_AUG), _f32),
                       pltpu.VMEM((CH, D_AUG), _f32),
                       pltpu.SemaphoreType.DMA,
                       pltpu.SemaphoreType.DMA],
        compiler_params=sc_params,
    )
    pa, pb = agg1(xa, eidx, z144)

    grid1 = (N_PAD // RB,)
    ha, hb, rinv = pl.pallas_call(
        _l1_body,
        grid=grid1,
        in_specs=[
            pl.BlockSpec((RB, D_AUG), lambda i: (i, 0)),
            pl.BlockSpec((RB, D_AUG), lambda i: (i, 0)),
            pl.BlockSpec((RB, D_FEAT), lambda i: (i, 0)),
            pl.BlockSpec((D_FEAT, D_HID), lambda i: (0, 0)),
            pl.BlockSpec((D_FEAT, D_HID), lambda i: (0, 0)),
            pl.BlockSpec((1, D_HID), lambda i: (0, 0)),
        ],
        out_specs=[
            pl.BlockSpec((RB, D_FEAT), lambda i: (i, 0)),
            pl.BlockSpec((RB, D_FEAT), lambda i: (i, 0)),
            pl.BlockSpec((RB, 1), lambda i: (i, 0)),
        ],
        out_shape=[jax.ShapeDtypeStruct((N_PAD, D_FEAT), _f32),
                   jax.ShapeDtypeStruct((N_PAD, D_FEAT), _f32),
                   jax.ShapeDtypeStruct((N_PAD, 1), _f32)],
    )(pa, pb, xp, W1l.T, W1r.T, b1l[None, :])

    agg2 = pl.kernel(
        _agg2_body,
        out_type=[jax.ShapeDtypeStruct((N_PAD, D_FEAT), _f32),
                  jax.ShapeDtypeStruct((N_PAD, D_FEAT), _f32)],
        mesh=mesh,
        scratch_types=[pltpu.VMEM_SHARED((N_PAD, D_FEAT), _f32),
                       pltpu.VMEM((2, CH), jnp.int32),
                       pltpu.VMEM((2, CH), jnp.int32),
                       pltpu.VMEM((CH, D_FEAT), _f32),
                       pltpu.VMEM((CH, D_FEAT), _f32),
                       pltpu.SemaphoreType.DMA,
                       pltpu.SemaphoreType.DMA],
        compiler_params=sc_params,
    )
    s2a, s2b = agg2(ha, hb, eidx, z128)

    w2lT = W2l.T
    w2rT = W2r.T
    out = pl.pallas_call(
        _l2_body,
        grid=grid1,
        in_specs=[
            pl.BlockSpec((RB, D_FEAT), lambda i: (i, 0)),
            pl.BlockSpec((RB, D_FEAT), lambda i: (i, 0)),
            pl.BlockSpec((RB, D_FEAT), lambda i: (i, 0)),
            pl.BlockSpec((RB, D_FEAT), lambda i: (i, 0)),
            pl.BlockSpec((RB, 1), lambda i: (i, 0)),
            pl.BlockSpec((D_FEAT, D_HID), lambda i: (0, 0)),
            pl.BlockSpec((D_FEAT, D_HID), lambda i: (0, 0)),
            pl.BlockSpec((D_FEAT, D_HID), lambda i: (0, 0)),
            pl.BlockSpec((D_FEAT, D_HID), lambda i: (0, 0)),
            pl.BlockSpec((1, D_HID), lambda i: (0, 0)),
            pl.BlockSpec((D_HID, 1), lambda i: (0, 0)),
            pl.BlockSpec((1, 1), lambda i: (0, 0)),
        ],
        out_specs=[pl.BlockSpec((RB, 1), lambda i: (i, 0))],
        out_shape=[jax.ShapeDtypeStruct((N_PAD, 1), _f32)],
    )(s2a, s2b, ha, hb, rinv,
      w2lT[:D_FEAT], w2lT[D_FEAT:], w2rT[:D_FEAT], w2rT[D_FEAT:],
      b2l[None, :], Wfc.T, bfc[None, :])[0]

    return out[:N_NODES]
